# Initial kernel scaffold; baseline (speedup 1.0000x reference)
import jax
import jax.numpy as jnp
from jax.experimental import pallas as pl


def kernel(ctx_embeddings, ctx_begin_pos, input_ids, token_table, pos_table):
    raise NotImplementedError("write your pallas kernel here")


# speedup vs baseline: 3.6070x; 3.6070x over previous
import jax
import jax.numpy as jnp
from jax import lax
from jax.experimental import pallas as pl
from jax.experimental.pallas import tpu as pltpu
from jax.experimental.pallas import tpu_sc as plsc

NC = 2
NS = 16
NW = NC * NS


def _sc_embed(ctx_embeddings, cbp, input_ids, token_table, pos_table):
    B, C, D = ctx_embeddings.shape
    _, S = input_ids.shape
    L = S + C
    BW = B // NW
    nd = D // 16

    mesh = plsc.VectorSubcoreMesh(core_axis_name="c", subcore_axis_name="s")

    @pl.kernel(
        out_type=jax.ShapeDtypeStruct((B, L, D), jnp.float32),
        mesh=mesh,
        scratch_types=[
            pltpu.VMEM((S,), jnp.int32),
            pltpu.VMEM((L,), jnp.int32),
            pltpu.VMEM((L, D), jnp.float32),
            pltpu.VMEM((L, D), jnp.float32),
            pltpu.VMEM((C, D), jnp.float32),
            pltpu.VMEM((BW + 16,), jnp.int32),
            pltpu.SemaphoreType.DMA,
        ],
        compiler_params=pltpu.CompilerParams(
            needs_layout_passes=False, use_tc_tiling_on_sc=False),
    )
    def k(ctx_hbm, cbp_hbm, ids_hbm, tok_hbm, pos_hbm, out_hbm,
          ids_v, gidx_v, rows_v, pos_v, ctx_v, cbp_v, sem):
        wid = lax.axis_index("s") * NC + lax.axis_index("c")
        base = wid * BW
        pltpu.sync_copy(pos_hbm, pos_v)
        pltpu.sync_copy(cbp_hbm.at[pl.ds(base, BW)], cbp_v.at[pl.ds(0, BW)])

        def sample(i, carry):
            bb = base + i
            my_cbp = cbp_v[pl.ds(i, 16)][0]
            pltpu.sync_copy(ids_hbm.at[bb], ids_v)
            pltpu.sync_copy(ctx_hbm.at[bb], ctx_v)

            for ch in range((L + 15) // 16):
                j = lax.iota(jnp.int32, 16) + ch * 16
                tj = jnp.where(j < my_cbp, j, j - C)
                tj = jnp.clip(tj, 0, S - 1)
                vals = plsc.load_gather(ids_v, [tj])
                if (ch + 1) * 16 <= L:
                    gidx_v[pl.ds(ch * 16, 16)] = vals
                else:
                    plsc.store_scatter(gidx_v, [j], vals, mask=j < L)

            pltpu.async_copy(tok_hbm.at[gidx_v], rows_v, sem).wait()

            def addrow(r, c2):
                for kk in range(nd):
                    sl = pl.ds(kk * 16, 16)
                    rows_v[r, sl] = rows_v[r, sl] + pos_v[r, sl]
                return c2
            lax.fori_loop(0, L, addrow, 0, unroll=False)

            def ctxrow(jj, c2):
                r = my_cbp + jj
                for kk in range(nd):
                    sl = pl.ds(kk * 16, 16)
                    rows_v[r, sl] = ctx_v[jj, sl] + pos_v[r, sl]
                return c2
            lax.fori_loop(0, C, ctxrow, 0, unroll=False)

            pltpu.sync_copy(rows_v, out_hbm.at[bb])
            return carry

        lax.fori_loop(0, BW, sample, 0, unroll=False)

    return k(ctx_embeddings, cbp, input_ids, token_table, pos_table)


def _causal_mask(B, L, dtype):
    neg = jnp.finfo(dtype).min
    BB = 8

    def body(o_ref):
        r = lax.broadcasted_iota(jnp.int32, (L, L), 0)
        c = lax.broadcasted_iota(jnp.int32, (L, L), 1)
        m = jnp.where(c > r, neg, jnp.zeros((), dtype)).astype(dtype)
        o_ref[...] = jnp.broadcast_to(m[None, None], (BB, 1, L, L))

    return pl.pallas_call(
        body,
        out_shape=jax.ShapeDtypeStruct((B, 1, L, L), dtype),
        grid=(B // BB,),
        out_specs=pl.BlockSpec((BB, 1, L, L), lambda i: (i, 0, 0, 0)),
    )()


def kernel(ctx_embeddings, ctx_begin_pos, input_ids, token_table, pos_table):
    B, C, D = ctx_embeddings.shape
    _, S = input_ids.shape
    L = S + C
    cbp = ctx_begin_pos.astype(jnp.int32)
    emb = _sc_embed(ctx_embeddings, cbp, input_ids.astype(jnp.int32),
                    token_table, pos_table)
    mask = _causal_mask(B, L, emb.dtype)
    return emb, mask


# speedup vs baseline: 4.0768x; 1.1303x over previous
import jax
import jax.numpy as jnp
from jax import lax
from jax.experimental import pallas as pl
from jax.experimental.pallas import tpu as pltpu
from jax.experimental.pallas import tpu_sc as plsc

NC = 2
NS = 16
NW = NC * NS


def _sc_embed(ctx_embeddings, cbp, input_ids, token_table, pos_table):
    B, C, D = ctx_embeddings.shape
    _, S = input_ids.shape
    L = S + C
    BW = B // NW
    nd = D // 16

    mesh = plsc.VectorSubcoreMesh(core_axis_name="c", subcore_axis_name="s")

    @pl.kernel(
        out_type=jax.ShapeDtypeStruct((L, B, D), jnp.float32),
        mesh=mesh,
        scratch_types=[
            pltpu.VMEM((S,), jnp.int32),
            pltpu.VMEM((L,), jnp.int32),
            pltpu.VMEM((L, D), jnp.float32),
            pltpu.VMEM((L, D), jnp.float32),
            pltpu.VMEM((C, D), jnp.float32),
            pltpu.VMEM((BW + 16,), jnp.int32),
            pltpu.SemaphoreType.DMA,
        ],
        compiler_params=pltpu.CompilerParams(
            needs_layout_passes=False, use_tc_tiling_on_sc=False),
    )
    def k(ctx_hbm, cbp_hbm, ids_hbm, tok_hbm, pos_hbm, out_hbm,
          ids_v, gidx_v, rows_v, pos_v, ctx_v, cbp_v, sem):
        wid = lax.axis_index("s") * NC + lax.axis_index("c")
        base = wid * BW
        pltpu.sync_copy(pos_hbm, pos_v)
        pltpu.sync_copy(cbp_hbm.at[pl.ds(base, BW)], cbp_v.at[pl.ds(0, BW)])

        def sample(i, carry):
            bb = base + i
            my_cbp = cbp_v[pl.ds(i, 16)][0]
            pltpu.sync_copy(ids_hbm.at[bb], ids_v)
            pltpu.sync_copy(ctx_hbm.at[bb], ctx_v)

            for ch in range((L + 15) // 16):
                j = lax.iota(jnp.int32, 16) + ch * 16
                tj = jnp.where(j < my_cbp, j, j - C)
                tj = jnp.clip(tj, 0, S - 1)
                vals = plsc.load_gather(ids_v, [tj])
                if (ch + 1) * 16 <= L:
                    gidx_v[pl.ds(ch * 16, 16)] = vals
                else:
                    plsc.store_scatter(gidx_v, [j], vals, mask=j < L)

            pltpu.async_copy(tok_hbm.at[gidx_v], rows_v, sem).wait()

            def addrow(r, c2):
                for kk in range(nd):
                    sl = pl.ds(kk * 16, 16)
                    rows_v[r, sl] = rows_v[r, sl] + pos_v[r, sl]
                return c2
            lax.fori_loop(0, L, addrow, 0, unroll=False)

            def ctxrow(jj, c2):
                r = my_cbp + jj
                for kk in range(nd):
                    sl = pl.ds(kk * 16, 16)
                    rows_v[r, sl] = ctx_v[jj, sl] + pos_v[r, sl]
                return c2
            lax.fori_loop(0, C, ctxrow, 0, unroll=False)

            pltpu.sync_copy(rows_v, out_hbm.at[:, bb])
            return carry

        lax.fori_loop(0, BW, sample, 0, unroll=False)

    return k(ctx_embeddings, cbp, input_ids, token_table, pos_table)


def kernel(ctx_embeddings, ctx_begin_pos, input_ids, token_table, pos_table):
    B, C, D = ctx_embeddings.shape
    _, S = input_ids.shape
    L = S + C
    cbp = ctx_begin_pos.astype(jnp.int32)
    emb3 = _sc_embed(ctx_embeddings, cbp, input_ids.astype(jnp.int32),
                     token_table, pos_table)
    emb = jnp.transpose(emb3, (1, 0, 2))
    neg = jnp.finfo(emb.dtype).min
    r = lax.broadcasted_iota(jnp.int32, (L, L), 0)
    c = lax.broadcasted_iota(jnp.int32, (L, L), 1)
    m = jnp.where(c > r, neg, jnp.zeros((), emb.dtype))
    mask = jnp.broadcast_to(m[None, None], (B, 1, L, L))
    return emb, mask


# speedup vs baseline: 4.7994x; 1.1773x over previous
import jax
import jax.numpy as jnp
from jax import lax
from jax.experimental import pallas as pl
from jax.experimental.pallas import tpu as pltpu
from jax.experimental.pallas import tpu_sc as plsc

NC = 2
NS = 16
NW = NC * NS


def _sc_embed(ctx5, cbp, input_ids, tok_sub, pos_table, B, C, D, S):
    L = S + C
    LP = (L + 15) // 16 * 16
    BW = B // NW
    ND = D // 128
    CT = C // 8

    mesh = plsc.VectorSubcoreMesh(core_axis_name="c", subcore_axis_name="s")

    @pl.kernel(
        out_type=jax.ShapeDtypeStruct((L, B // 8, ND, 8, 128), jnp.float32),
        mesh=mesh,
        scratch_types=[
            pltpu.VMEM((S,), jnp.int32),
            pltpu.VMEM((ND, LP), jnp.int32),
            pltpu.VMEM((ND, LP, 128), jnp.float32),
            pltpu.VMEM((L, D), jnp.float32),
            pltpu.VMEM((BW + 16,), jnp.int32),
            pltpu.SemaphoreType.DMA,
        ],
        compiler_params=pltpu.CompilerParams(
            needs_layout_passes=False, use_tc_tiling_on_sc=False),
    )
    def k(ctx_hbm, cbp_hbm, ids_hbm, tok_hbm, pos_hbm, out_hbm,
          ids_v, gidx_v, rows_v, pos_v, cbp_v, sem):
        wid = lax.axis_index("s") * NC + lax.axis_index("c")
        base = wid * BW
        pltpu.sync_copy(pos_hbm, pos_v)
        pltpu.sync_copy(cbp_hbm.at[pl.ds(base, BW)], cbp_v.at[pl.ds(0, BW)])

        def sample(i, carry):
            bb = base + i
            my_cbp = cbp_v[pl.ds(i, 16)][0]
            pltpu.sync_copy(ids_hbm.at[bb], ids_v)

            for ch in range(LP // 16):
                j = lax.iota(jnp.int32, 16) + ch * 16
                tj = jnp.where(j < my_cbp, j, j - C)
                tj = jnp.clip(tj, 0, S - 1)
                t = plsc.load_gather(ids_v, [tj])
                sub = (t >> 3) * (ND * 8) + (t & 7)
                for dt in range(ND):
                    gidx_v[dt, pl.ds(ch * 16, 16)] = sub + dt * 8

            gcps = [pltpu.async_copy(tok_hbm.at[gidx_v.at[dt]], rows_v.at[dt],
                                     sem) for dt in range(ND)]
            for cp in gcps:
                cp.wait()

            ccps = [pltpu.async_copy(
                        ctx_hbm.at[bb, rt, dt],
                        rows_v.at[dt, pl.ds(my_cbp + rt * 8, 8)], sem)
                    for rt in range(CT) for dt in range(ND)]
            for cp in ccps:
                cp.wait()

            def addrow(r, c2):
                for dt in range(ND):
                    for kk in range(8):
                        rows_v[dt, r, pl.ds(kk * 16, 16)] = (
                            rows_v[dt, r, pl.ds(kk * 16, 16)]
                            + pos_v[r, pl.ds(dt * 128 + kk * 16, 16)])
                return c2
            lax.fori_loop(0, L, addrow, 0, unroll=False)

            bt = bb // 8
            bs = bb % 8
            wcps = [pltpu.async_copy(rows_v.at[dt, pl.ds(0, L)],
                                     out_hbm.at[:, bt, dt, bs], sem)
                    for dt in range(ND)]
            for cp in wcps:
                cp.wait()
            return carry

        lax.fori_loop(0, BW, sample, 0, unroll=False)

    return k(ctx5, cbp, input_ids, tok_sub, pos_table)


def kernel(ctx_embeddings, ctx_begin_pos, input_ids, token_table, pos_table):
    B, C, D = ctx_embeddings.shape
    V, _ = token_table.shape
    _, S = input_ids.shape
    L = S + C
    ND = D // 128
    cbp = ctx_begin_pos.astype(jnp.int32)
    tok_sub = jnp.transpose(
        jnp.reshape(token_table, (V // 8, 8, ND, 128)), (0, 2, 1, 3)
    ).reshape(V // 8 * ND * 8, 128)
    ctx5 = jnp.transpose(
        jnp.reshape(ctx_embeddings, (B, C // 8, 8, ND, 128)), (0, 1, 3, 2, 4))
    emb5 = _sc_embed(ctx5, cbp, input_ids.astype(jnp.int32), tok_sub,
                     pos_table, B, C, D, S)
    emb = jnp.transpose(emb5, (1, 3, 0, 2, 4)).reshape(B, L, D)
    neg = jnp.finfo(emb.dtype).min
    r = lax.broadcasted_iota(jnp.int32, (L, L), 0)
    c = lax.broadcasted_iota(jnp.int32, (L, L), 1)
    m = jnp.where(c > r, neg, jnp.zeros((), emb.dtype))
    mask = jnp.broadcast_to(m[None, None], (B, 1, L, L))
    return emb, mask


# speedup vs baseline: 5.4749x; 1.1407x over previous
import jax
import jax.numpy as jnp
from jax import lax
from jax.experimental import pallas as pl
from jax.experimental.pallas import tpu as pltpu
from jax.experimental.pallas import tpu_sc as plsc

NC = 2
NS = 16
NW = NC * NS


def _sc_embed(ctx5, cbp, input_ids, tok_sub, pos_table, B, C, D, S):
    L = S + C
    LP = (L + 15) // 16 * 16
    NSLAB = NW // 2
    BW = B // NSLAB
    ND = D // 128
    NH = ND // 2
    CT = C // 8

    mesh = plsc.VectorSubcoreMesh(core_axis_name="c", subcore_axis_name="s")

    @pl.kernel(
        out_type=jax.ShapeDtypeStruct((L, B // 8, ND, 8, 128), jnp.float32),
        mesh=mesh,
        scratch_types=[
            pltpu.VMEM((2, S), jnp.int32),
            pltpu.VMEM((2, NH, LP), jnp.int32),
            pltpu.VMEM((2, NH, LP, 128), jnp.float32),
            pltpu.VMEM((2, CT, NH, 8, 128), jnp.float32),
            pltpu.VMEM((L, NH * 128), jnp.float32),
            pltpu.VMEM((BW + 16,), jnp.int32),
            pltpu.SemaphoreType.DMA,
            pltpu.SemaphoreType.DMA,
            pltpu.SemaphoreType.DMA,
            pltpu.SemaphoreType.DMA,
            pltpu.SemaphoreType.DMA,
            pltpu.SemaphoreType.DMA,
            pltpu.SemaphoreType.DMA,
            pltpu.SemaphoreType.DMA,
        ],
        compiler_params=pltpu.CompilerParams(
            needs_layout_passes=False, use_tc_tiling_on_sc=False),
    )
    def k(ctx_hbm, cbp_hbm, ids_hbm, tok_hbm, pos_hbm, out_hbm,
          ids_v, gidx_v, rows_v, ctx_v, pos_v, cbp_v,
          sem_i0, sem_i1, sem_g0, sem_g1, sem_c0, sem_c1, sem_w0, sem_w1):
        wid = lax.axis_index("s") * NC + lax.axis_index("c")
        slab = wid // 2
        half = wid % 2
        dt0 = half * NH
        base = slab * BW
        sem_i = [sem_i0, sem_i1]
        sem_g = [sem_g0, sem_g1]
        sem_c = [sem_c0, sem_c1]
        sem_w = [sem_w0, sem_w1]

        pltpu.sync_copy(pos_hbm.at[:, pl.ds(dt0 * 128, NH * 128)], pos_v)
        pltpu.sync_copy(cbp_hbm.at[pl.ds(base, BW)], cbp_v.at[pl.ds(0, BW)])

        def cbp_of(i):
            return cbp_v[pl.ds(i, 16)][0]

        def build_gidx(p, my_cbp):
            for ch in range(LP // 16):
                j = lax.iota(jnp.int32, 16) + ch * 16
                tj = jnp.where(j < my_cbp, j, j - C)
                tj = jnp.clip(tj, 0, S - 1)
                t = plsc.load_gather(ids_v.at[p], [tj])
                sub = (t >> 3) * (ND * 8) + (t & 7)
                for d in range(NH):
                    gidx_v[p, d, pl.ds(ch * 16, 16)] = sub + (dt0 + d) * 8

        def fire_ids(p, i):
            pltpu.async_copy(ids_hbm.at[base + i], ids_v.at[p], sem_i[p])

        def wait_ids(p):
            pltpu.make_async_copy(ids_hbm.at[0], ids_v.at[p], sem_i[p]).wait()

        def fire_gc(p, i):
            bb = base + i
            for d in range(NH):
                pltpu.async_copy(tok_hbm.at[gidx_v.at[p, d]],
                                 rows_v.at[p, d], sem_g[p])
            pltpu.async_copy(
                ctx_hbm.at[bb, :, pl.ds(dt0, NH)], ctx_v.at[p], sem_c[p])

        def wait_gc(p):
            for d in range(NH):
                pltpu.make_async_copy(tok_hbm.at[gidx_v.at[p, d]],
                                      rows_v.at[p, d], sem_g[p]).wait()
            pltpu.make_async_copy(
                ctx_hbm.at[0, :, pl.ds(dt0, NH)], ctx_v.at[p], sem_c[p]).wait()

        def fire_w(p, i):
            bb = base + i
            bt = bb // 8
            bs = bb % 8
            for d in range(NH):
                pltpu.async_copy(rows_v.at[p, d, pl.ds(0, L)],
                                 out_hbm.at[:, bt, dt0 + d, bs], sem_w[p])

        def wait_w(p):
            for d in range(NH):
                pltpu.make_async_copy(rows_v.at[p, d, pl.ds(0, L)],
                                      out_hbm.at[:, 0, dt0 + d, 0],
                                      sem_w[p]).wait()

        def add_pass(p, my_cbp):
            def addrow(r, c2):
                in_ctx = jnp.logical_and(r >= my_cbp, r < my_cbp + C)
                jj = r - my_cbp

                @pl.when(in_ctx)
                def _():
                    for d in range(NH):
                        for kk in range(8):
                            sl = pl.ds(kk * 16, 16)
                            rows_v[p, d, r, sl] = (
                                ctx_v[p, jj // 8, d, jj % 8, sl]
                                + pos_v[r, pl.ds(d * 128 + kk * 16, 16)])

                @pl.when(jnp.logical_not(in_ctx))
                def _():
                    for d in range(NH):
                        for kk in range(8):
                            sl = pl.ds(kk * 16, 16)
                            rows_v[p, d, r, sl] = (
                                rows_v[p, d, r, sl]
                                + pos_v[r, pl.ds(d * 128 + kk * 16, 16)])
                return c2
            lax.fori_loop(0, L, addrow, 0, unroll=False)

        fire_ids(0, 0)
        wait_ids(0)
        build_gidx(0, cbp_of(0))
        fire_gc(0, 0)
        fire_ids(1, 1)

        def body(h, carry):
            i = 2 * h
            wait_gc(0)
            wait_ids(1)
            build_gidx(1, cbp_of(i + 1))

            @pl.when(h > 0)
            def _():
                wait_w(1)
            fire_gc(1, i + 1)

            @pl.when(h < BW // 2 - 1)
            def _():
                fire_ids(0, i + 2)
            add_pass(0, cbp_of(i))
            fire_w(0, i)

            wait_gc(1)

            @pl.when(h < BW // 2 - 1)
            def _():
                wait_ids(0)
                build_gidx(0, cbp_of(i + 2))
                wait_w(0)
                fire_gc(0, i + 2)
                fire_ids(1, i + 3)
            add_pass(1, cbp_of(i + 1))
            fire_w(1, i + 1)
            return carry

        lax.fori_loop(0, BW // 2, body, 0, unroll=False)
        wait_w(0)
        wait_w(1)

    return k(ctx5, cbp, input_ids, tok_sub, pos_table)


def kernel(ctx_embeddings, ctx_begin_pos, input_ids, token_table, pos_table):
    B, C, D = ctx_embeddings.shape
    V, _ = token_table.shape
    _, S = input_ids.shape
    L = S + C
    ND = D // 128
    cbp = ctx_begin_pos.astype(jnp.int32)
    tok_sub = jnp.transpose(
        jnp.reshape(token_table, (V // 8, 8, ND, 128)), (0, 2, 1, 3)
    ).reshape(V // 8 * ND * 8, 128)
    ctx5 = jnp.transpose(
        jnp.reshape(ctx_embeddings, (B, C // 8, 8, ND, 128)), (0, 1, 3, 2, 4))
    emb5 = _sc_embed(ctx5, cbp, input_ids.astype(jnp.int32), tok_sub,
                     pos_table, B, C, D, S)
    emb = jnp.transpose(emb5, (1, 3, 0, 2, 4)).reshape(B, L, D)
    neg = jnp.finfo(emb.dtype).min
    r = lax.broadcasted_iota(jnp.int32, (L, L), 0)
    c = lax.broadcasted_iota(jnp.int32, (L, L), 1)
    m = jnp.where(c > r, neg, jnp.zeros((), emb.dtype))
    mask = jnp.broadcast_to(m[None, None], (B, 1, L, L))
    return emb, mask


# speedup vs baseline: 7.4382x; 1.3586x over previous
import jax
import jax.numpy as jnp
from jax import lax
from jax.experimental import pallas as pl
from jax.experimental.pallas import tpu as pltpu
from jax.experimental.pallas import tpu_sc as plsc

NC = 2
NS = 16
NW = NC * NS


def _sc_embed(ctx5, cbp, input_ids, tok_sub, pos_table, B, C, D, S):
    L = S + C
    LP = (L + 15) // 16 * 16
    NSLAB = NW // 2
    BW = B // NSLAB
    ND = D // 128
    NH = ND // 2
    CT = C // 8

    mesh = plsc.VectorSubcoreMesh(core_axis_name="c", subcore_axis_name="s")

    @pl.kernel(
        out_type=jax.ShapeDtypeStruct((L, B // 8, ND, 8, 128), jnp.float32),
        mesh=mesh,
        scratch_types=[
            pltpu.VMEM((2, S), jnp.int32),
            pltpu.VMEM((2, NH, LP), jnp.int32),
            pltpu.VMEM((2, NH, LP, 128), jnp.float32),
            pltpu.VMEM((2, CT, NH, 8, 128), jnp.float32),
            pltpu.VMEM((NH, LP, 128), jnp.float32),
            pltpu.VMEM((BW + 16,), jnp.int32),
            pltpu.SemaphoreType.DMA,
            pltpu.SemaphoreType.DMA,
            pltpu.SemaphoreType.DMA,
            pltpu.SemaphoreType.DMA,
            pltpu.SemaphoreType.DMA,
            pltpu.SemaphoreType.DMA,
            pltpu.SemaphoreType.DMA,
            pltpu.SemaphoreType.DMA,
        ],
        compiler_params=pltpu.CompilerParams(
            needs_layout_passes=False, use_tc_tiling_on_sc=False),
    )
    def k(ctx_hbm, cbp_hbm, ids_hbm, tok_hbm, pos_hbm, out_hbm,
          ids_v, gidx_v, rows_v, ctx_v, pos_v, cbp_v,
          sem_i0, sem_i1, sem_g0, sem_g1, sem_c0, sem_c1, sem_w0, sem_w1):
        wid = lax.axis_index("s") * NC + lax.axis_index("c")
        slab = wid // 2
        half = wid % 2
        dt0 = half * NH
        base = slab * BW
        sem_i = [sem_i0, sem_i1]
        sem_g = [sem_g0, sem_g1]
        sem_c = [sem_c0, sem_c1]
        sem_w = [sem_w0, sem_w1]

        for d in range(NH):
            pltpu.sync_copy(pos_hbm.at[:, pl.ds((dt0 + d) * 128, 128)],
                            pos_v.at[d, pl.ds(0, L)])
        pltpu.sync_copy(cbp_hbm.at[pl.ds(base, BW)], cbp_v.at[pl.ds(0, BW)])

        def cbp_of(i):
            return cbp_v[pl.ds(i, 16)][0]

        def build_gidx(p, my_cbp):
            for ch in range(LP // 16):
                j = lax.iota(jnp.int32, 16) + ch * 16
                tj = jnp.where(j < my_cbp, j, j - C)
                tj = jnp.clip(tj, 0, S - 1)
                t = plsc.load_gather(ids_v.at[p], [tj])
                sub = (t >> 3) * (ND * 8) + (t & 7)
                for d in range(NH):
                    gidx_v[p, d, pl.ds(ch * 16, 16)] = sub + (dt0 + d) * 8

        def fire_ids(p, i):
            pltpu.async_copy(ids_hbm.at[base + i], ids_v.at[p], sem_i[p])

        def wait_ids(p):
            pltpu.make_async_copy(ids_hbm.at[0], ids_v.at[p], sem_i[p]).wait()

        def fire_gc(p, i):
            bb = base + i
            for d in range(NH):
                pltpu.async_copy(tok_hbm.at[gidx_v.at[p, d]],
                                 rows_v.at[p, d], sem_g[p])
            pltpu.async_copy(
                ctx_hbm.at[bb, :, pl.ds(dt0, NH)], ctx_v.at[p], sem_c[p])

        def wait_gc(p):
            for d in range(NH):
                pltpu.make_async_copy(tok_hbm.at[gidx_v.at[p, d]],
                                      rows_v.at[p, d], sem_g[p]).wait()
            pltpu.make_async_copy(
                ctx_hbm.at[0, :, pl.ds(dt0, NH)], ctx_v.at[p], sem_c[p]).wait()

        def fire_w(p, i):
            bb = base + i
            bt = bb // 8
            bs = bb % 8
            for d in range(NH):
                pltpu.async_copy(rows_v.at[p, d, pl.ds(0, L)],
                                 out_hbm.at[:, bt, dt0 + d, bs], sem_w[p])

        def wait_w(p):
            for d in range(NH):
                pltpu.make_async_copy(rows_v.at[p, d, pl.ds(0, L)],
                                      out_hbm.at[:, 0, dt0 + d, 0],
                                      sem_w[p]).wait()

        def add_pass(p, my_cbp):
            for d in range(NH):
                def addrow(r, c2, d=d):
                    for kk in range(8):
                        sl = pl.ds(kk * 16, 16)
                        rows_v[p, d, r, sl] = (rows_v[p, d, r, sl]
                                               + pos_v[d, r, sl])
                    return c2
                lax.fori_loop(0, LP, addrow, 0, unroll=4)

            def ctxrow(jj, c2):
                r = my_cbp + jj
                for d in range(NH):
                    for kk in range(8):
                        sl = pl.ds(kk * 16, 16)
                        rows_v[p, d, r, sl] = (
                            ctx_v[p, jj // 8, d, jj % 8, sl] + pos_v[d, r, sl])
                return c2
            lax.fori_loop(0, C, ctxrow, 0, unroll=False)

        fire_ids(0, 0)
        wait_ids(0)
        build_gidx(0, cbp_of(0))
        fire_gc(0, 0)
        fire_ids(1, 1)

        def body(h, carry):
            i = 2 * h
            wait_gc(0)
            wait_ids(1)
            build_gidx(1, cbp_of(i + 1))

            @pl.when(h > 0)
            def _():
                wait_w(1)
            fire_gc(1, i + 1)

            @pl.when(h < BW // 2 - 1)
            def _():
                fire_ids(0, i + 2)
            add_pass(0, cbp_of(i))
            fire_w(0, i)

            wait_gc(1)

            @pl.when(h < BW // 2 - 1)
            def _():
                wait_ids(0)
                build_gidx(0, cbp_of(i + 2))
                wait_w(0)
                fire_gc(0, i + 2)
                fire_ids(1, i + 3)
            add_pass(1, cbp_of(i + 1))
            fire_w(1, i + 1)
            return carry

        lax.fori_loop(0, BW // 2, body, 0, unroll=False)
        wait_w(0)
        wait_w(1)

    return k(ctx5, cbp, input_ids, tok_sub, pos_table)


def kernel(ctx_embeddings, ctx_begin_pos, input_ids, token_table, pos_table):
    B, C, D = ctx_embeddings.shape
    V, _ = token_table.shape
    _, S = input_ids.shape
    L = S + C
    ND = D // 128
    cbp = ctx_begin_pos.astype(jnp.int32)
    tok_sub = jnp.transpose(
        jnp.reshape(token_table, (V // 8, 8, ND, 128)), (0, 2, 1, 3)
    ).reshape(V // 8 * ND * 8, 128)
    ctx5 = jnp.transpose(
        jnp.reshape(ctx_embeddings, (B, C // 8, 8, ND, 128)), (0, 1, 3, 2, 4))
    emb5 = _sc_embed(ctx5, cbp, input_ids.astype(jnp.int32), tok_sub,
                     pos_table, B, C, D, S)
    emb = jnp.transpose(emb5, (1, 3, 0, 2, 4)).reshape(B, L, D)
    neg = jnp.finfo(emb.dtype).min
    r = lax.broadcasted_iota(jnp.int32, (L, L), 0)
    c = lax.broadcasted_iota(jnp.int32, (L, L), 1)
    m = jnp.where(c > r, neg, jnp.zeros((), emb.dtype))
    mask = jnp.broadcast_to(m[None, None], (B, 1, L, L))
    return emb, mask
